# baseline (speedup 1.0000x reference)
import functools

import numpy as np
import jax
import jax.numpy as jnp
from jax import lax
from jax.experimental import pallas as pl
from jax.experimental.pallas import tpu as pltpu
from jax.experimental.pallas import tpu_sc as plsc


def _positional_encoding(sequence_length, embedding_depth):
    half = embedding_depth / 2
    positions = np.arange(sequence_length)[:, np.newaxis]
    depths = np.arange(half)[np.newaxis, :] / half
    angle_rates = 1 / 10000 ** depths
    angle_rads = positions * angle_rates
    enc = np.concatenate([np.sin(angle_rads), np.cos(angle_rads)], axis=-1)
    return enc.astype(np.float32)


_B, _T, _D = 4096, 200, 64
_CH = 128
_NB = 2
_NW = 32
_NCHUNK = (_B * _T) // (_CH * _NW)
_GROUPS = _NCHUNK // _NB
_LANES = 16


def _build():
    mesh = plsc.VectorSubcoreMesh(core_axis_name="c", subcore_axis_name="s")
    out_type = jax.ShapeDtypeStruct((_B * _T, _D), jnp.float32)
    scratch = [
        pltpu.VMEM((_NCHUNK, _CH), jnp.int32),
        pltpu.VMEM((_T, _D), jnp.float32),
        pltpu.VMEM((_NB, _CH), jnp.int32),
    ]
    scratch += [pltpu.VMEM((_CH, 2 * _D), jnp.float32) for _ in range(_NB)]
    scratch += [pltpu.VMEM((_CH, _D), jnp.float32) for _ in range(_NB)]
    scratch += [pltpu.SemaphoreType.DMA] * (2 * _NB)

    @functools.partial(pl.kernel, mesh=mesh, out_type=out_type,
                       scratch_types=scratch,
                       compiler_params=pltpu.CompilerParams(
                           use_tc_tiling_on_sc=True,
                           needs_layout_passes=False))
    def k(xf, tp, pos, out, idx_v, pos_v, hidx, *rest):
        gb = rest[0:_NB]
        ob = rest[_NB:2 * _NB]
        gsem = rest[2 * _NB:3 * _NB]
        wsem = rest[3 * _NB:4 * _NB]

        wid = lax.axis_index("s") * 2 + lax.axis_index("c")
        crow0 = wid * _NCHUNK
        out0 = wid * _NCHUNK * _CH

        pltpu.sync_copy(xf.at[pl.ds(crow0, _NCHUNK)], idx_v)
        pltpu.sync_copy(pos, pos_v)

        lanes = lax.iota(jnp.int32, _LANES)

        def start_gather(j, b):
            for v in range(_CH // _LANES):
                sl = pl.ds(v * _LANES, _LANES)
                hidx[b, sl] = lax.shift_right_logical(idx_v[j, sl], 1)
            pltpu.make_async_copy(tp.at[hidx.at[b]], gb[b], gsem[b]).start()

        def gather_wait(b):
            pltpu.make_async_copy(tp.at[hidx.at[b]], gb[b], gsem[b]).wait()

        def write(j, b):
            dst = out.at[pl.ds(out0 + j * _CH, _CH)]
            return pltpu.make_async_copy(ob[b], dst, wsem[b])

        for b in range(_NB):
            start_gather(b, b)

        def group(gi, carry):
            for b in range(_NB):
                j = gi * _NB + b
                gather_wait(b)

                @pl.when(gi > 0)
                def _():
                    write(j - _NB, b).wait()

                prow0 = lax.rem(j * _CH, _T)

                for rb in range(_CH // _LANES):
                    sl = pl.ds(rb * _LANES, _LANES)
                    pvv = (idx_v[j, sl] & 1) * _D

                    def add_row(i, c):
                        r = rb * _LANES + i
                        p = jnp.sum(jnp.where(lanes == i, pvv, 0))
                        pr = prow0 + r
                        pr = jnp.where(pr >= _T, pr - _T, pr)
                        for d in range(_D // _LANES):
                            osl = pl.ds(d * _LANES, _LANES)
                            gsl = pl.ds(p + d * _LANES, _LANES)
                            ob[b][r, osl] = gb[b][r, gsl] + pos_v[pr, osl]
                        return c

                    lax.fori_loop(0, _LANES, add_row, 0)

                write(j, b).start()

                @pl.when(j + _NB < _NCHUNK)
                def _():
                    start_gather(j + _NB, b)
            return carry

        lax.fori_loop(0, _GROUPS, group, 0)
        for b in range(_NB):
            write(_NCHUNK - _NB + b, b).wait()

    return k


_KERNEL = _build()


def kernel(x, table):
    xf = x.astype(jnp.int32).reshape(_B * _T // _CH, _CH)
    tp = table.reshape(500000, 2 * _D)
    pos = jnp.asarray(_positional_encoding(_T, _D))
    out = _KERNEL(xf, tp, pos)
    return out.reshape(_B, _T, _D)


# speedup vs baseline: 1.0146x; 1.0146x over previous
import functools

import numpy as np
import jax
import jax.numpy as jnp
from jax import lax
from jax.experimental import pallas as pl
from jax.experimental.pallas import tpu as pltpu
from jax.experimental.pallas import tpu_sc as plsc


def _positional_encoding(sequence_length, embedding_depth):
    half = embedding_depth / 2
    positions = np.arange(sequence_length)[:, np.newaxis]
    depths = np.arange(half)[np.newaxis, :] / half
    angle_rates = 1 / 10000 ** depths
    angle_rads = positions * angle_rates
    enc = np.concatenate([np.sin(angle_rads), np.cos(angle_rads)], axis=-1)
    return enc.astype(np.float32)


_B, _T, _D = 4096, 200, 64
_CH = 128
_NB = 2
_NW = 32
_NCHUNK = (_B * _T) // (_CH * _NW)
_GROUPS = _NCHUNK // _NB
_LANES = 16


def _build():
    mesh = plsc.VectorSubcoreMesh(core_axis_name="c", subcore_axis_name="s")
    out_type = jax.ShapeDtypeStruct((_B * _T, _D), jnp.float32)
    scratch = [
        pltpu.VMEM((_NCHUNK, _CH), jnp.int32),
        pltpu.VMEM((_T, _D), jnp.float32),
        pltpu.VMEM((_NB, _CH), jnp.int32),
    ]
    scratch += [pltpu.VMEM((_CH, 2 * _D), jnp.float32) for _ in range(_NB)]
    scratch += [pltpu.VMEM((_CH, _D), jnp.float32) for _ in range(_NB)]
    scratch += [pltpu.SemaphoreType.DMA] * (2 * _NB)

    @functools.partial(pl.kernel, mesh=mesh, out_type=out_type,
                       scratch_types=scratch,
                       compiler_params=pltpu.CompilerParams(
                           use_tc_tiling_on_sc=True,
                           needs_layout_passes=False))
    def k(xf, tp, pos, out, idx_v, pos_v, hidx, *rest):
        gb = rest[0:_NB]
        ob = rest[_NB:2 * _NB]
        gsem = rest[2 * _NB:3 * _NB]
        wsem = rest[3 * _NB:4 * _NB]

        wid = lax.axis_index("s") * 2 + lax.axis_index("c")
        crow0 = wid * _NCHUNK
        out0 = wid * _NCHUNK * _CH

        pltpu.sync_copy(xf.at[pl.ds(crow0, _NCHUNK)], idx_v)
        pltpu.sync_copy(pos, pos_v)

        lanes = lax.iota(jnp.int32, _LANES)

        def start_gather(j, b):
            for v in range(_CH // _LANES):
                sl = pl.ds(v * _LANES, _LANES)
                hidx[b, sl] = lax.shift_right_logical(idx_v[j, sl], 1)
            pltpu.make_async_copy(tp.at[hidx.at[b]], gb[b], gsem[b]).start()

        def gather_wait(b):
            pltpu.make_async_copy(tp.at[hidx.at[b]], gb[b], gsem[b]).wait()

        def write(j, b):
            dst = out.at[pl.ds(out0 + j * _CH, _CH)]
            return pltpu.make_async_copy(ob[b], dst, wsem[b])

        for b in range(_NB):
            start_gather(b, b)

        def group(gi, carry):
            for b in range(_NB):
                j = gi * _NB + b
                gather_wait(b)

                @pl.when(gi > 0)
                def _():
                    write(j - _NB, b).wait()

                prow0 = lax.rem(j * _CH, _T)

                for rb in range(_CH // _LANES):
                    sl = pl.ds(rb * _LANES, _LANES)
                    pvv = idx_v[j, sl] & 1

                    def add_row(i, c):
                        r = rb * _LANES + i
                        pb = lax.gather(
                            pvv, (lanes * 0 + i)[:, None],
                            lax.GatherDimensionNumbers(
                                offset_dims=(), collapsed_slice_dims=(0,),
                                start_index_map=(0,)),
                            (1,),
                            mode=lax.GatherScatterMode.PROMISE_IN_BOUNDS)
                        hi_mask = pb > 0
                        pr = prow0 + r
                        pr = jnp.where(pr >= _T, pr - _T, pr)
                        for d in range(_D // _LANES):
                            osl = pl.ds(d * _LANES, _LANES)
                            lo = gb[b][r, osl]
                            hi = gb[b][r, pl.ds(_D + d * _LANES, _LANES)]
                            val = jnp.where(hi_mask, hi, lo)
                            ob[b][r, osl] = val + pos_v[pr, osl]
                        return c

                    lax.fori_loop(0, _LANES, add_row, 0)

                write(j, b).start()

                @pl.when(j + _NB < _NCHUNK)
                def _():
                    start_gather(j + _NB, b)
            return carry

        lax.fori_loop(0, _GROUPS, group, 0)
        for b in range(_NB):
            write(_NCHUNK - _NB + b, b).wait()

    return k


_KERNEL = _build()


def kernel(x, table):
    xf = x.astype(jnp.int32).reshape(_B * _T // _CH, _CH)
    tp = table.reshape(500000, 2 * _D)
    pos = jnp.asarray(_positional_encoding(_T, _D))
    out = _KERNEL(xf, tp, pos)
    return out.reshape(_B, _T, _D)


# speedup vs baseline: 1.0817x; 1.0662x over previous
import functools

import numpy as np
import jax
import jax.numpy as jnp
from jax import lax
from jax.experimental import pallas as pl
from jax.experimental.pallas import tpu as pltpu
from jax.experimental.pallas import tpu_sc as plsc


def _positional_encoding(sequence_length, embedding_depth):
    half = embedding_depth / 2
    positions = np.arange(sequence_length)[:, np.newaxis]
    depths = np.arange(half)[np.newaxis, :] / half
    angle_rates = 1 / 10000 ** depths
    angle_rads = positions * angle_rates
    enc = np.concatenate([np.sin(angle_rads), np.cos(angle_rads)], axis=-1)
    return enc.astype(np.float32)


_B, _T, _D = 4096, 200, 64
_CH = 128
_NB = 2
_NW = 32
_NCHUNK = (_B * _T) // (_CH * _NW)
_GROUPS = _NCHUNK // _NB
_LANES = 16


def _build():
    mesh = plsc.VectorSubcoreMesh(core_axis_name="c", subcore_axis_name="s")
    out_type = jax.ShapeDtypeStruct((_B * _T, _D), jnp.float32)
    scratch = [
        pltpu.VMEM((_NCHUNK, _CH), jnp.int32),
        pltpu.VMEM((_T, _D), jnp.float32),
        pltpu.VMEM((_NB, _CH), jnp.int32),
    ]
    scratch += [pltpu.VMEM((_CH, 2 * _D), jnp.float32) for _ in range(_NB)]
    scratch += [pltpu.VMEM((_CH, _D), jnp.float32) for _ in range(_NB)]
    scratch += [pltpu.SemaphoreType.DMA] * (2 * _NB)

    @functools.partial(pl.kernel, mesh=mesh, out_type=out_type,
                       scratch_types=scratch,
                       compiler_params=pltpu.CompilerParams(
                           use_tc_tiling_on_sc=True,
                           needs_layout_passes=False))
    def k(xf, tp, pos, out, idx_v, pos_v, hidx, *rest):
        gb = rest[0:_NB]
        ob = rest[_NB:2 * _NB]
        gsem = rest[2 * _NB:3 * _NB]
        wsem = rest[3 * _NB:4 * _NB]

        wid = lax.axis_index("s") * 2 + lax.axis_index("c")
        crow0 = wid * _NCHUNK
        out0 = wid * _NCHUNK * _CH

        pltpu.sync_copy(xf.at[pl.ds(crow0, _NCHUNK)], idx_v)
        pltpu.sync_copy(pos, pos_v)

        lanes = lax.iota(jnp.int32, _LANES)

        def start_gather(j, b):
            for v in range(_CH // _LANES):
                sl = pl.ds(v * _LANES, _LANES)
                hidx[b, sl] = lax.shift_right_logical(idx_v[j, sl], 1)
            pltpu.make_async_copy(tp.at[hidx.at[b]], gb[b], gsem[b]).start()

        def gather_wait(b):
            pltpu.make_async_copy(tp.at[hidx.at[b]], gb[b], gsem[b]).wait()

        def write(j, b):
            dst = out.at[pl.ds(out0 + j * _CH, _CH)]
            return pltpu.make_async_copy(ob[b], dst, wsem[b])

        for b in range(_NB):
            start_gather(b, b)

        def group(gi, carry):
            for b in range(_NB):
                j = gi * _NB + b
                gather_wait(b)

                @pl.when(gi > 0)
                def _():
                    write(j - _NB, b).wait()

                prow0 = lax.rem(j * _CH, _T)

                dn = lax.GatherDimensionNumbers(
                    offset_dims=(), collapsed_slice_dims=(0,),
                    start_index_map=(0,))

                def rb_body(rb, c):
                    r0 = rb * _LANES
                    sl = pl.ds(r0, _LANES)
                    pvv = idx_v[j, sl] & 1
                    for i in range(_LANES):
                        r = r0 + i
                        pb = lax.gather(
                            pvv, (lanes * 0 + i)[:, None], dn, (1,),
                            mode=lax.GatherScatterMode.PROMISE_IN_BOUNDS)
                        hi_mask = pb > 0
                        pr = prow0 + r
                        pr = jnp.where(pr >= _T, pr - _T, pr)
                        for d in range(_D // _LANES):
                            osl = pl.ds(d * _LANES, _LANES)
                            lo = gb[b][r, osl]
                            hi = gb[b][r, pl.ds(_D + d * _LANES, _LANES)]
                            val = jnp.where(hi_mask, hi, lo)
                            ob[b][r, osl] = val + pos_v[pr, osl]
                    return c

                lax.fori_loop(0, _CH // _LANES, rb_body, 0)

                write(j, b).start()

                @pl.when(j + _NB < _NCHUNK)
                def _():
                    start_gather(j + _NB, b)
            return carry

        lax.fori_loop(0, _GROUPS, group, 0)
        for b in range(_NB):
            write(_NCHUNK - _NB + b, b).wait()

    return k


_KERNEL = _build()


def kernel(x, table):
    xf = x.astype(jnp.int32).reshape(_B * _T // _CH, _CH)
    tp = table.reshape(500000, 2 * _D)
    pos = jnp.asarray(_positional_encoding(_T, _D))
    out = _KERNEL(xf, tp, pos)
    return out.reshape(_B, _T, _D)


# speedup vs baseline: 1.1289x; 1.0436x over previous
import functools

import numpy as np
import jax
import jax.numpy as jnp
from jax import lax
from jax.experimental import pallas as pl
from jax.experimental.pallas import tpu as pltpu
from jax.experimental.pallas import tpu_sc as plsc


def _positional_encoding(sequence_length, embedding_depth):
    half = embedding_depth / 2
    positions = np.arange(sequence_length)[:, np.newaxis]
    depths = np.arange(half)[np.newaxis, :] / half
    angle_rates = 1 / 10000 ** depths
    angle_rads = positions * angle_rates
    enc = np.concatenate([np.sin(angle_rads), np.cos(angle_rads)], axis=-1)
    return enc.astype(np.float32)


_B, _T, _D = 4096, 200, 64
_CHUNK = 100
_NB = 4
_NW = 32
_NCHUNK = (_B * _T) // (_CHUNK * _NW)
_GROUPS = _NCHUNK // _NB
_LANES = 16


def _build():
    mesh = plsc.VectorSubcoreMesh(core_axis_name="c", subcore_axis_name="s")
    out_type = jax.ShapeDtypeStruct((_B * _T * _D,), jnp.float32)
    scratch = [
        pltpu.VMEM((_NCHUNK, _CHUNK), jnp.int32),
        pltpu.VMEM((_T, _D), jnp.float32),
    ]
    scratch += [pltpu.VMEM((_CHUNK, _D), jnp.float32) for _ in range(_NB)]
    scratch += [pltpu.VMEM((_CHUNK * _D,), jnp.float32) for _ in range(_NB)]
    scratch += [pltpu.SemaphoreType.DMA] * (2 * _NB)

    @functools.partial(pl.kernel, mesh=mesh, out_type=out_type,
                       scratch_types=scratch,
                       compiler_params=pltpu.CompilerParams(
                           use_tc_tiling_on_sc=False))
    def k(xr, table, pos, out, idx_v, pos_v, *rest):
        gb = rest[0:_NB]
        ob = rest[_NB:2 * _NB]
        gsem = rest[2 * _NB:3 * _NB]
        wsem = rest[3 * _NB:4 * _NB]

        wid = lax.axis_index("s") * 2 + lax.axis_index("c")
        row0 = wid * _NCHUNK
        out0 = wid * _NCHUNK * _CHUNK * _D

        pltpu.sync_copy(xr.at[pl.ds(row0, _NCHUNK)], idx_v)
        pltpu.sync_copy(pos, pos_v)

        def gather(j, b):
            return pltpu.make_async_copy(table.at[idx_v.at[j]], gb[b], gsem[b])

        def write(j, b):
            dst = out.at[pl.ds(out0 + j * _CHUNK * _D, _CHUNK * _D)]
            return pltpu.make_async_copy(ob[b], dst, wsem[b])

        for b in range(_NB):
            gather(b, b).start()

        def group(gi, carry):
            for b in range(_NB):
                j = gi * _NB + b
                gather(j, b).wait()

                @pl.when(gi > 0)
                def _():
                    write(j - _NB, b).wait()

                prow0 = (j % 2) * _CHUNK

                def add_row(r, c):
                    pr = prow0 + r
                    for d in range(_D // _LANES):
                        sl = pl.ds(d * _LANES, _LANES)
                        osl = pl.ds(r * _D + d * _LANES, _LANES)
                        ob[b][osl] = gb[b][r, sl] + pos_v[pr, sl]
                    return c

                lax.fori_loop(0, _CHUNK, add_row, 0)
                write(j, b).start()

                @pl.when(j + _NB < _NCHUNK)
                def _():
                    gather(j + _NB, b).start()
            return carry

        lax.fori_loop(0, _GROUPS, group, 0)
        for b in range(_NB):
            write(_NCHUNK - _NB + b, b).wait()

    return k


_KERNEL = _build()


def kernel(x, table):
    xr = x.reshape(_B * _T // _CHUNK, _CHUNK).astype(jnp.int32)
    pos = jnp.asarray(_positional_encoding(_T, _D))
    out = _KERNEL(xr, table, pos)
    return out.reshape(_B, _T, _D)
